# baseline (speedup 1.0000x reference)
import jax
import jax.numpy as jnp
import numpy as np
from jax import lax
from jax.experimental import pallas as pl
from jax.experimental.pallas import tpu as pltpu
from jax.experimental.pallas import tpu_sc as plsc

_N = 256
_F = 3 * 224 * 224
_C = 768
_S = _F // _C
_NSUB = _N * _S
_NC, _NS = 2, 16
_NW = _NC * _NS
_RW = _N // _NW
_SUBS = _RW * _S
_TILE = 16
_TI = _SUBS // _TILE
_LANES = 16
_KC = _C // _LANES

_PERM = np.asarray([
    121, 35, 130, 148, 197, 45, 176, 179, 139, 188, 99, 144, 152, 189, 31, 112,
    85, 63, 117, 174, 114, 254, 82, 65, 7, 4, 101, 102, 78, 163, 157, 183,
    29, 240, 177, 108, 83, 129, 212, 44, 211, 16, 58, 123, 37, 111, 19, 61,
    2, 142, 34, 156, 5, 90, 175, 167, 251, 110, 72, 155, 178, 219, 153, 30,
    42, 186, 246, 3, 70, 67, 223, 39, 56, 192, 169, 218, 195, 173, 245, 241,
    69, 80, 22, 6, 199, 118, 235, 54, 77, 147, 18, 249, 10, 11, 234, 53,
    236, 94, 32, 217, 159, 15, 184, 49, 137, 50, 138, 20, 237, 253, 185, 43,
    92, 8, 140, 233, 24, 81, 239, 96, 154, 135, 160, 106, 128, 191, 9, 200,
    40, 187, 71, 248, 164, 207, 93, 59, 201, 158, 210, 75, 131, 97, 66, 25,
    196, 242, 206, 243, 238, 73, 13, 52, 203, 202, 255, 194, 88, 250, 62, 230,
    150, 209, 132, 87, 76, 198, 60, 244, 47, 33, 79, 180, 247, 14, 228, 17,
    38, 86, 231, 190, 232, 23, 105, 220, 0, 145, 213, 226, 133, 41, 64, 21,
    161, 166, 124, 116, 26, 165, 168, 193, 57, 208, 181, 89, 146, 182, 126, 125,
    1, 115, 28, 113, 225, 172, 162, 48, 170, 227, 36, 252, 119, 151, 120, 224,
    122, 100, 91, 222, 55, 103, 51, 215, 127, 98, 107, 27, 74, 136, 229, 204,
    221, 12, 134, 109, 84, 205, 171, 143, 68, 216, 149, 141, 104, 95, 214, 46,
], dtype=np.int32)

_BIDX = (_PERM[:, None] * _S + np.arange(_S, dtype=np.int32)[None, :]
         ).reshape(-1).astype(np.int32)


def _body(x_hbm, bidx_hbm, lam_hbm, lab_hbm, perm_hbm, out_hbm, olab_hbm,
          bidx_v, lam_v, a0, a1, b0, b1, o0, o1, perm_v, olab_v,
          sa0, sa1, sb0, sb1, so0, so1, sl):
    cid = lax.axis_index("c")
    sid = lax.axis_index("s")
    wid = sid * _NC + cid
    base = wid * _SUBS

    A = (a0, a1)
    B = (b0, b1)
    O = (o0, o1)
    SA = (sa0, sa1)
    SB = (sb0, sb1)
    SO = (so0, so1)

    pltpu.sync_copy(bidx_hbm.at[pl.ds(base, _SUBS)], bidx_v)
    pltpu.sync_copy(lam_hbm, lam_v)
    lam = lam_v[...]

    def a_copy(t, p):
        return pltpu.make_async_copy(
            x_hbm.at[pl.ds(base + t * _TILE, _TILE)], A[p], SA[p])

    def b_copy(t, p):
        return pltpu.make_async_copy(
            x_hbm.at[bidx_v.at[pl.ds(t * _TILE, _TILE)]], B[p], SB[p])

    def o_copy(t, p):
        return pltpu.make_async_copy(
            O[p], out_hbm.at[pl.ds(base + t * _TILE, _TILE)], SO[p])

    @pl.when(wid == 0)
    def _():
        pltpu.sync_copy(perm_hbm, perm_v)
        for h in range(2):
            pltpu.async_copy(
                lab_hbm.at[perm_v.at[pl.ds(h * 128, 128)]],
                olab_v.at[pl.ds(h * 128, 128)], sl).wait()
        pltpu.sync_copy(olab_v, olab_hbm)

    a_copy(0, 0).start()
    b_copy(0, 0).start()
    a_copy(1, 1).start()
    b_copy(1, 1).start()

    @pl.loop(0, _TI, step=2)
    def _(g):
        for p in range(2):
            t = g + p

            a_copy(t, p).wait()
            b_copy(t, p).wait()

            @pl.when(t >= 2)
            def _():
                o_copy(t - 2, p).wait()

            @pl.loop(0, _TILE)
            def _(r):
                @pl.loop(0, _KC)
                def _(k):
                    av = A[p][r, pl.ds(k * _LANES, _LANES)]
                    bv = B[p][r, pl.ds(k * _LANES, _LANES)]
                    O[p][r, pl.ds(k * _LANES, _LANES)] = av + lam * (bv - av)

            o_copy(t, p).start()

            @pl.when(t + 2 < _TI)
            def _():
                a_copy(t + 2, p).start()
                b_copy(t + 2, p).start()

    o_copy(_TI - 2, 0).wait()
    o_copy(_TI - 1, 1).wait()


import functools


@functools.cache
def _mix_call():
    return pl.kernel(
        _body,
    out_type=[
        jax.ShapeDtypeStruct((_NSUB, _C), jnp.float32),
        jax.ShapeDtypeStruct((_N, 128), jnp.int32),
    ],
    mesh=plsc.VectorSubcoreMesh(core_axis_name="c", subcore_axis_name="s",
                                num_cores=_NC, num_subcores=_NS),
    scratch_types=[
        pltpu.VMEM((_SUBS,), jnp.int32),
        pltpu.VMEM((_LANES,), jnp.float32),
        pltpu.VMEM((_TILE, _C), jnp.float32),
        pltpu.VMEM((_TILE, _C), jnp.float32),
        pltpu.VMEM((_TILE, _C), jnp.float32),
        pltpu.VMEM((_TILE, _C), jnp.float32),
        pltpu.VMEM((_TILE, _C), jnp.float32),
        pltpu.VMEM((_TILE, _C), jnp.float32),
        pltpu.VMEM((_N,), jnp.int32),
        pltpu.VMEM((_N, 128), jnp.int32),
        pltpu.SemaphoreType.DMA,
        pltpu.SemaphoreType.DMA,
        pltpu.SemaphoreType.DMA,
        pltpu.SemaphoreType.DMA,
        pltpu.SemaphoreType.DMA,
        pltpu.SemaphoreType.DMA,
        pltpu.SemaphoreType.DMA,
    ],
)


def kernel(x, labels, lambda_):
    x2d = x.reshape(_NSUB, _C)
    lam16 = jnp.full((_LANES,), lambda_, dtype=jnp.float32)
    lab2d = jnp.broadcast_to(labels.astype(jnp.int32)[:, None], (_N, 128))
    out2d, olab = _mix_call()(x2d, jnp.asarray(_BIDX), lam16, lab2d,
                              jnp.asarray(_PERM))
    return out2d.reshape(x.shape), labels, olab[:, 0].astype(labels.dtype)


# speedup vs baseline: 1.8297x; 1.8297x over previous
import functools

import jax
import jax.numpy as jnp
import numpy as np
from jax import lax
from jax.experimental import pallas as pl
from jax.experimental.pallas import tpu as pltpu
from jax.experimental.pallas import tpu_sc as plsc

_N = 256
_CH = 3
_H = 224
_W = 224
_NC, _NS = 2, 16
_NW = _NC * _NS
_RW = _N // _NW
_VS = 56
_NV = _H // _VS
_CPR = _CH * _NV
_TI = _RW * _CPR
_LANES = 16
_KW = _W // _LANES

_PERM = np.asarray([
    121, 35, 130, 148, 197, 45, 176, 179, 139, 188, 99, 144, 152, 189, 31, 112,
    85, 63, 117, 174, 114, 254, 82, 65, 7, 4, 101, 102, 78, 163, 157, 183,
    29, 240, 177, 108, 83, 129, 212, 44, 211, 16, 58, 123, 37, 111, 19, 61,
    2, 142, 34, 156, 5, 90, 175, 167, 251, 110, 72, 155, 178, 219, 153, 30,
    42, 186, 246, 3, 70, 67, 223, 39, 56, 192, 169, 218, 195, 173, 245, 241,
    69, 80, 22, 6, 199, 118, 235, 54, 77, 147, 18, 249, 10, 11, 234, 53,
    236, 94, 32, 217, 159, 15, 184, 49, 137, 50, 138, 20, 237, 253, 185, 43,
    92, 8, 140, 233, 24, 81, 239, 96, 154, 135, 160, 106, 128, 191, 9, 200,
    40, 187, 71, 248, 164, 207, 93, 59, 201, 158, 210, 75, 131, 97, 66, 25,
    196, 242, 206, 243, 238, 73, 13, 52, 203, 202, 255, 194, 88, 250, 62, 230,
    150, 209, 132, 87, 76, 198, 60, 244, 47, 33, 79, 180, 247, 14, 228, 17,
    38, 86, 231, 190, 232, 23, 105, 220, 0, 145, 213, 226, 133, 41, 64, 21,
    161, 166, 124, 116, 26, 165, 168, 193, 57, 208, 181, 89, 146, 182, 126, 125,
    1, 115, 28, 113, 225, 172, 162, 48, 170, 227, 36, 252, 119, 151, 120, 224,
    122, 100, 91, 222, 55, 103, 51, 215, 127, 98, 107, 27, 74, 136, 229, 204,
    221, 12, 134, 109, 84, 205, 171, 143, 68, 216, 149, 141, 104, 95, 214, 46,
], dtype=np.int32)


def _body(x_hbm, lam_hbm, lab_hbm, perm_hbm, out_hbm, olab_hbm,
          lam_v, a0, a1, b0, b1, o0, o1, perm_v, olab_v,
          sa0, sa1, sb0, sb1, so0, so1, sl):
    cid = lax.axis_index("c")
    sid = lax.axis_index("s")
    wid = sid * _NC + cid

    A = (a0, a1)
    B = (b0, b1)
    O = (o0, o1)
    SA = (sa0, sa1)
    SB = (sb0, sb1)
    SO = (so0, so1)

    pltpu.sync_copy(lam_hbm, lam_v)
    lam = lam_v[...]

    zero = jnp.int32(0)
    pvals = []
    for q in range(_RW):
        v = zero
        for w in range(_NW):
            v = v + jnp.where(wid == w, jnp.int32(int(_PERM[_RW * w + q])), zero)
        pvals.append(v)

    def decomp(t):
        q = t // _CPR
        sub = t % _CPR
        ch = sub // _NV
        v0 = (sub % _NV) * _VS
        pr = zero
        for i in range(_RW):
            pr = pr + jnp.where(q == i, pvals[i], zero)
        return q, ch, v0, pr

    def a_copy(t, p):
        q, ch, v0, _ = decomp(t)
        r = wid * _RW + q
        return pltpu.make_async_copy(
            x_hbm.at[r, ch, pl.ds(v0, _VS), :], A[p], SA[p])

    def b_copy(t, p):
        _, ch, v0, pr = decomp(t)
        return pltpu.make_async_copy(
            x_hbm.at[pr, ch, pl.ds(v0, _VS), :], B[p], SB[p])

    def o_copy(t, p):
        q, ch, v0, _ = decomp(t)
        r = wid * _RW + q
        return pltpu.make_async_copy(
            O[p], out_hbm.at[r, ch, pl.ds(v0, _VS), :], SO[p])

    @pl.when(wid == 0)
    def _():
        pltpu.sync_copy(perm_hbm, perm_v)
        for h in range(4):
            pltpu.async_copy(
                lab_hbm.at[perm_v.at[pl.ds(h * 64, 64)]], olab_v, sl).wait()
            pltpu.sync_copy(olab_v, olab_hbm.at[pl.ds(h * 64, 64)])

    a_copy(0, 0).start()
    b_copy(0, 0).start()
    a_copy(1, 1).start()
    b_copy(1, 1).start()

    @pl.loop(0, _TI, step=2)
    def _(g):
        for p in range(2):
            t = g + p

            a_copy(t, p).wait()
            b_copy(t, p).wait()

            @pl.when(t >= 2)
            def _():
                o_copy(t - 2, p).wait()

            @pl.loop(0, _VS)
            def _(rr):
                for k in range(_KW):
                    av = A[p][rr, pl.ds(k * _LANES, _LANES)]
                    bv = B[p][rr, pl.ds(k * _LANES, _LANES)]
                    O[p][rr, pl.ds(k * _LANES, _LANES)] = av + lam * (bv - av)

            o_copy(t, p).start()

            @pl.when(t + 2 < _TI)
            def _():
                a_copy(t + 2, p).start()
                b_copy(t + 2, p).start()

    o_copy(_TI - 2, 0).wait()
    o_copy(_TI - 1, 1).wait()


@functools.cache
def _mix_call():
    return pl.kernel(
        _body,
        out_type=[
            jax.ShapeDtypeStruct((_N, _CH, _H, _W), jnp.float32),
            jax.ShapeDtypeStruct((_N, 128), jnp.int32),
        ],
        mesh=plsc.VectorSubcoreMesh(core_axis_name="c", subcore_axis_name="s",
                                    num_cores=_NC, num_subcores=_NS),
        scratch_types=[
            pltpu.VMEM((_LANES,), jnp.float32),
            pltpu.VMEM((_VS, _W), jnp.float32),
            pltpu.VMEM((_VS, _W), jnp.float32),
            pltpu.VMEM((_VS, _W), jnp.float32),
            pltpu.VMEM((_VS, _W), jnp.float32),
            pltpu.VMEM((_VS, _W), jnp.float32),
            pltpu.VMEM((_VS, _W), jnp.float32),
            pltpu.VMEM((_N,), jnp.int32),
            pltpu.VMEM((64, 128), jnp.int32),
            pltpu.SemaphoreType.DMA,
            pltpu.SemaphoreType.DMA,
            pltpu.SemaphoreType.DMA,
            pltpu.SemaphoreType.DMA,
            pltpu.SemaphoreType.DMA,
            pltpu.SemaphoreType.DMA,
            pltpu.SemaphoreType.DMA,
        ],
    )


def kernel(x, labels, lambda_):
    lam16 = jnp.full((_LANES,), lambda_, dtype=jnp.float32)
    lab2d = jnp.broadcast_to(labels.astype(jnp.int32)[:, None], (_N, 128))
    out, olab = _mix_call()(x, lam16, lab2d, jnp.asarray(_PERM))
    return out, labels, olab[:, 0].astype(labels.dtype)


# speedup vs baseline: 7.6484x; 4.1802x over previous
import functools

import jax
import jax.numpy as jnp
import numpy as np
from jax import lax
from jax.experimental import pallas as pl
from jax.experimental.pallas import tpu as pltpu
from jax.experimental.pallas import tpu_sc as plsc

_N = 256
_R = 3 * 224 * 224
_BR = 3136
_NC, _NS = 2, 16

_PERM = np.asarray([
    121, 35, 130, 148, 197, 45, 176, 179, 139, 188, 99, 144, 152, 189, 31, 112,
    85, 63, 117, 174, 114, 254, 82, 65, 7, 4, 101, 102, 78, 163, 157, 183,
    29, 240, 177, 108, 83, 129, 212, 44, 211, 16, 58, 123, 37, 111, 19, 61,
    2, 142, 34, 156, 5, 90, 175, 167, 251, 110, 72, 155, 178, 219, 153, 30,
    42, 186, 246, 3, 70, 67, 223, 39, 56, 192, 169, 218, 195, 173, 245, 241,
    69, 80, 22, 6, 199, 118, 235, 54, 77, 147, 18, 249, 10, 11, 234, 53,
    236, 94, 32, 217, 159, 15, 184, 49, 137, 50, 138, 20, 237, 253, 185, 43,
    92, 8, 140, 233, 24, 81, 239, 96, 154, 135, 160, 106, 128, 191, 9, 200,
    40, 187, 71, 248, 164, 207, 93, 59, 201, 158, 210, 75, 131, 97, 66, 25,
    196, 242, 206, 243, 238, 73, 13, 52, 203, 202, 255, 194, 88, 250, 62, 230,
    150, 209, 132, 87, 76, 198, 60, 244, 47, 33, 79, 180, 247, 14, 228, 17,
    38, 86, 231, 190, 232, 23, 105, 220, 0, 145, 213, 226, 133, 41, 64, 21,
    161, 166, 124, 116, 26, 165, 168, 193, 57, 208, 181, 89, 146, 182, 126, 125,
    1, 115, 28, 113, 225, 172, 162, 48, 170, 227, 36, 252, 119, 151, 120, 224,
    122, 100, 91, 222, 55, 103, 51, 215, 127, 98, 107, 27, 74, 136, 229, 204,
    221, 12, 134, 109, 84, 205, 171, 143, 68, 216, 149, 141, 104, 95, 214, 46,
], dtype=np.int32)

_P_ONEHOT = np.zeros((_N, _N), dtype=np.float32)
_P_ONEHOT[_PERM, np.arange(_N)] = 1.0


def _tc_body(y_ref, m_ref, o_ref):
    o_ref[...] = jnp.dot(y_ref[...], m_ref[...],
                         preferred_element_type=jnp.float32)


@functools.cache
def _tc_call():
    return pl.pallas_call(
        _tc_body,
        grid=(_R // _BR,),
        in_specs=[
            pl.BlockSpec((_BR, _N), lambda i: (i, 0)),
            pl.BlockSpec((_N, _N), lambda i: (0, 0)),
        ],
        out_specs=pl.BlockSpec((_BR, _N), lambda i: (i, 0)),
        out_shape=jax.ShapeDtypeStruct((_R, _N), jnp.float32),
    )


def _sc_body(lab_hbm, perm_hbm, olab_hbm, perm_v, olab_v, sl):
    cid = lax.axis_index("c")
    sid = lax.axis_index("s")
    wid = sid * _NC + cid

    @pl.when(wid == 0)
    def _():
        pltpu.sync_copy(perm_hbm, perm_v)
        for h in range(4):
            pltpu.async_copy(
                lab_hbm.at[perm_v.at[pl.ds(h * 64, 64)]], olab_v, sl).wait()
            pltpu.sync_copy(olab_v, olab_hbm.at[pl.ds(h * 64, 64)])


@functools.cache
def _sc_call():
    return pl.kernel(
        _sc_body,
        out_type=[jax.ShapeDtypeStruct((_N, 128), jnp.int32)],
        mesh=plsc.VectorSubcoreMesh(core_axis_name="c", subcore_axis_name="s",
                                    num_cores=_NC, num_subcores=_NS),
        scratch_types=[
            pltpu.VMEM((_N,), jnp.int32),
            pltpu.VMEM((64, 128), jnp.int32),
            pltpu.SemaphoreType.DMA,
        ],
    )


def kernel(x, labels, lambda_):
    y = x.transpose(1, 2, 3, 0).reshape(_R, _N)
    lam = lambda_.astype(jnp.float32)
    m = ((1.0 - lam) * jnp.eye(_N, dtype=jnp.float32)
         + lam * jnp.asarray(_P_ONEHOT))
    out2d = _tc_call()(y, m)
    out = out2d.reshape(3, 224, 224, _N).transpose(3, 0, 1, 2)

    lab2d = jnp.broadcast_to(labels.astype(jnp.int32)[:, None], (_N, 128))
    (olab,) = _sc_call()(lab2d, jnp.asarray(_PERM))
    return out, labels, olab[:, 0].astype(labels.dtype)


# speedup vs baseline: 7.7430x; 1.0124x over previous
import functools

import jax
import jax.numpy as jnp
import numpy as np
from jax import lax
from jax.experimental import pallas as pl
from jax.experimental.pallas import tpu as pltpu
from jax.experimental.pallas import tpu_sc as plsc

_N = 256
_R = 3 * 224 * 224
_BR = 3136
_NC, _NS = 2, 16

_PERM = np.asarray([
    121, 35, 130, 148, 197, 45, 176, 179, 139, 188, 99, 144, 152, 189, 31, 112,
    85, 63, 117, 174, 114, 254, 82, 65, 7, 4, 101, 102, 78, 163, 157, 183,
    29, 240, 177, 108, 83, 129, 212, 44, 211, 16, 58, 123, 37, 111, 19, 61,
    2, 142, 34, 156, 5, 90, 175, 167, 251, 110, 72, 155, 178, 219, 153, 30,
    42, 186, 246, 3, 70, 67, 223, 39, 56, 192, 169, 218, 195, 173, 245, 241,
    69, 80, 22, 6, 199, 118, 235, 54, 77, 147, 18, 249, 10, 11, 234, 53,
    236, 94, 32, 217, 159, 15, 184, 49, 137, 50, 138, 20, 237, 253, 185, 43,
    92, 8, 140, 233, 24, 81, 239, 96, 154, 135, 160, 106, 128, 191, 9, 200,
    40, 187, 71, 248, 164, 207, 93, 59, 201, 158, 210, 75, 131, 97, 66, 25,
    196, 242, 206, 243, 238, 73, 13, 52, 203, 202, 255, 194, 88, 250, 62, 230,
    150, 209, 132, 87, 76, 198, 60, 244, 47, 33, 79, 180, 247, 14, 228, 17,
    38, 86, 231, 190, 232, 23, 105, 220, 0, 145, 213, 226, 133, 41, 64, 21,
    161, 166, 124, 116, 26, 165, 168, 193, 57, 208, 181, 89, 146, 182, 126, 125,
    1, 115, 28, 113, 225, 172, 162, 48, 170, 227, 36, 252, 119, 151, 120, 224,
    122, 100, 91, 222, 55, 103, 51, 215, 127, 98, 107, 27, 74, 136, 229, 204,
    221, 12, 134, 109, 84, 205, 171, 143, 68, 216, 149, 141, 104, 95, 214, 46,
], dtype=np.int32)

_P_ONEHOT = np.zeros((_N, _N), dtype=np.float32)
_P_ONEHOT[_PERM, np.arange(_N)] = 1.0


def _tc_body(lam_ref, y_ref, p_ref, o_ref, m_ref):
    @pl.when(pl.program_id(0) == 0)
    def _():
        lam = lam_ref[0]
        row = lax.broadcasted_iota(jnp.int32, (_N, _N), 0)
        col = lax.broadcasted_iota(jnp.int32, (_N, _N), 1)
        eye = (row == col).astype(jnp.float32)
        m_ref[...] = (1.0 - lam) * eye + lam * p_ref[...]

    o_ref[...] = jnp.dot(y_ref[...], m_ref[...],
                         preferred_element_type=jnp.float32)


@functools.cache
def _tc_call():
    return pl.pallas_call(
        _tc_body,
        grid=(_R // _BR,),
        in_specs=[
            pl.BlockSpec(memory_space=pltpu.SMEM),
            pl.BlockSpec((_BR, _N), lambda i: (i, 0)),
            pl.BlockSpec((_N, _N), lambda i: (0, 0)),
        ],
        out_specs=pl.BlockSpec((_BR, _N), lambda i: (i, 0)),
        out_shape=jax.ShapeDtypeStruct((_R, _N), jnp.float32),
        scratch_shapes=[pltpu.VMEM((_N, _N), jnp.float32)],
    )


def _sc_body(lab_hbm, perm_hbm, olab_hbm, perm_v, olab_v, sl):
    cid = lax.axis_index("c")
    sid = lax.axis_index("s")
    wid = sid * _NC + cid

    @pl.when(wid == 0)
    def _():
        pltpu.sync_copy(perm_hbm, perm_v)
        for h in range(4):
            pltpu.async_copy(
                lab_hbm.at[perm_v.at[pl.ds(h * 64, 64)]], olab_v, sl).wait()
            pltpu.sync_copy(olab_v, olab_hbm.at[pl.ds(h * 64, 64)])


@functools.cache
def _sc_call():
    return pl.kernel(
        _sc_body,
        out_type=[jax.ShapeDtypeStruct((_N, 128), jnp.int32)],
        mesh=plsc.VectorSubcoreMesh(core_axis_name="c", subcore_axis_name="s",
                                    num_cores=_NC, num_subcores=_NS),
        scratch_types=[
            pltpu.VMEM((_N,), jnp.int32),
            pltpu.VMEM((64, 128), jnp.int32),
            pltpu.SemaphoreType.DMA,
        ],
    )


def kernel(x, labels, lambda_):
    y = x.transpose(1, 2, 3, 0).reshape(_R, _N)
    lam1 = lambda_.astype(jnp.float32).reshape(1)
    out2d = _tc_call()(lam1, y, jnp.asarray(_P_ONEHOT))
    out = out2d.reshape(3, 224, 224, _N).transpose(3, 0, 1, 2)

    lab2d = jnp.broadcast_to(labels.astype(jnp.int32)[:, None], (_N, 128))
    (olab,) = _sc_call()(lab2d, jnp.asarray(_PERM))
    return out, labels, olab[:, 0].astype(labels.dtype)


# speedup vs baseline: 8.0720x; 1.0425x over previous
import functools

import jax
import jax.numpy as jnp
import numpy as np
from jax import lax
from jax.experimental import pallas as pl
from jax.experimental.pallas import tpu as pltpu
from jax.experimental.pallas import tpu_sc as plsc

_N = 256
_R = 3 * 224 * 224
_BR = 6272
_NC, _NS = 2, 16

_PERM = np.asarray([
    121, 35, 130, 148, 197, 45, 176, 179, 139, 188, 99, 144, 152, 189, 31, 112,
    85, 63, 117, 174, 114, 254, 82, 65, 7, 4, 101, 102, 78, 163, 157, 183,
    29, 240, 177, 108, 83, 129, 212, 44, 211, 16, 58, 123, 37, 111, 19, 61,
    2, 142, 34, 156, 5, 90, 175, 167, 251, 110, 72, 155, 178, 219, 153, 30,
    42, 186, 246, 3, 70, 67, 223, 39, 56, 192, 169, 218, 195, 173, 245, 241,
    69, 80, 22, 6, 199, 118, 235, 54, 77, 147, 18, 249, 10, 11, 234, 53,
    236, 94, 32, 217, 159, 15, 184, 49, 137, 50, 138, 20, 237, 253, 185, 43,
    92, 8, 140, 233, 24, 81, 239, 96, 154, 135, 160, 106, 128, 191, 9, 200,
    40, 187, 71, 248, 164, 207, 93, 59, 201, 158, 210, 75, 131, 97, 66, 25,
    196, 242, 206, 243, 238, 73, 13, 52, 203, 202, 255, 194, 88, 250, 62, 230,
    150, 209, 132, 87, 76, 198, 60, 244, 47, 33, 79, 180, 247, 14, 228, 17,
    38, 86, 231, 190, 232, 23, 105, 220, 0, 145, 213, 226, 133, 41, 64, 21,
    161, 166, 124, 116, 26, 165, 168, 193, 57, 208, 181, 89, 146, 182, 126, 125,
    1, 115, 28, 113, 225, 172, 162, 48, 170, 227, 36, 252, 119, 151, 120, 224,
    122, 100, 91, 222, 55, 103, 51, 215, 127, 98, 107, 27, 74, 136, 229, 204,
    221, 12, 134, 109, 84, 205, 171, 143, 68, 216, 149, 141, 104, 95, 214, 46,
], dtype=np.int32)

_P_ONEHOT = np.zeros((_N, _N), dtype=np.float32)
_P_ONEHOT[_PERM, np.arange(_N)] = 1.0


def _tc_body(lam_ref, y_ref, p_ref, o_ref, m_ref):
    @pl.when(pl.program_id(0) == 0)
    def _():
        lam = lam_ref[0]
        row = lax.broadcasted_iota(jnp.int32, (_N, _N), 0)
        col = lax.broadcasted_iota(jnp.int32, (_N, _N), 1)
        eye = (row == col).astype(jnp.float32)
        m_ref[...] = (1.0 - lam) * eye + lam * p_ref[...]

    o_ref[...] = jnp.dot(y_ref[...], m_ref[...],
                         preferred_element_type=jnp.float32)


@functools.cache
def _tc_call():
    return pl.pallas_call(
        _tc_body,
        grid=(_R // _BR,),
        in_specs=[
            pl.BlockSpec(memory_space=pltpu.SMEM),
            pl.BlockSpec((_BR, _N), lambda i: (i, 0)),
            pl.BlockSpec((_N, _N), lambda i: (0, 0)),
        ],
        out_specs=pl.BlockSpec((_BR, _N), lambda i: (i, 0)),
        out_shape=jax.ShapeDtypeStruct((_R, _N), jnp.float32),
        scratch_shapes=[pltpu.VMEM((_N, _N), jnp.float32)],
    )


def _sc_body(lab_hbm, perm_hbm, olab_hbm, perm_v, olab_v, sl):
    cid = lax.axis_index("c")
    sid = lax.axis_index("s")
    wid = sid * _NC + cid

    @pl.when(wid == 0)
    def _():
        pltpu.sync_copy(perm_hbm, perm_v)
        for h in range(4):
            pltpu.async_copy(
                lab_hbm.at[perm_v.at[pl.ds(h * 64, 64)]], olab_v, sl).wait()
            pltpu.sync_copy(olab_v, olab_hbm.at[pl.ds(h * 64, 64)])


@functools.cache
def _sc_call():
    return pl.kernel(
        _sc_body,
        out_type=[jax.ShapeDtypeStruct((_N, 128), jnp.int32)],
        mesh=plsc.VectorSubcoreMesh(core_axis_name="c", subcore_axis_name="s",
                                    num_cores=_NC, num_subcores=_NS),
        scratch_types=[
            pltpu.VMEM((_N,), jnp.int32),
            pltpu.VMEM((64, 128), jnp.int32),
            pltpu.SemaphoreType.DMA,
        ],
    )


def kernel(x, labels, lambda_):
    y = x.transpose(1, 2, 3, 0).reshape(_R, _N)
    lam1 = lambda_.astype(jnp.float32).reshape(1)
    out2d = _tc_call()(lam1, y, jnp.asarray(_P_ONEHOT))
    out = out2d.reshape(3, 224, 224, _N).transpose(3, 0, 1, 2)

    lab2d = jnp.broadcast_to(labels.astype(jnp.int32)[:, None], (_N, 128))
    (olab,) = _sc_call()(lab2d, jnp.asarray(_PERM))
    return out, labels, olab[:, 0].astype(labels.dtype)


# speedup vs baseline: 8.1454x; 1.0091x over previous
import functools

import jax
import jax.numpy as jnp
import numpy as np
from jax import lax
from jax.experimental import pallas as pl
from jax.experimental.pallas import tpu as pltpu
from jax.experimental.pallas import tpu_sc as plsc

_N = 256
_R = 3 * 224 * 224
_BR = 9408
_NC, _NS = 2, 16

_PERM = np.asarray([
    121, 35, 130, 148, 197, 45, 176, 179, 139, 188, 99, 144, 152, 189, 31, 112,
    85, 63, 117, 174, 114, 254, 82, 65, 7, 4, 101, 102, 78, 163, 157, 183,
    29, 240, 177, 108, 83, 129, 212, 44, 211, 16, 58, 123, 37, 111, 19, 61,
    2, 142, 34, 156, 5, 90, 175, 167, 251, 110, 72, 155, 178, 219, 153, 30,
    42, 186, 246, 3, 70, 67, 223, 39, 56, 192, 169, 218, 195, 173, 245, 241,
    69, 80, 22, 6, 199, 118, 235, 54, 77, 147, 18, 249, 10, 11, 234, 53,
    236, 94, 32, 217, 159, 15, 184, 49, 137, 50, 138, 20, 237, 253, 185, 43,
    92, 8, 140, 233, 24, 81, 239, 96, 154, 135, 160, 106, 128, 191, 9, 200,
    40, 187, 71, 248, 164, 207, 93, 59, 201, 158, 210, 75, 131, 97, 66, 25,
    196, 242, 206, 243, 238, 73, 13, 52, 203, 202, 255, 194, 88, 250, 62, 230,
    150, 209, 132, 87, 76, 198, 60, 244, 47, 33, 79, 180, 247, 14, 228, 17,
    38, 86, 231, 190, 232, 23, 105, 220, 0, 145, 213, 226, 133, 41, 64, 21,
    161, 166, 124, 116, 26, 165, 168, 193, 57, 208, 181, 89, 146, 182, 126, 125,
    1, 115, 28, 113, 225, 172, 162, 48, 170, 227, 36, 252, 119, 151, 120, 224,
    122, 100, 91, 222, 55, 103, 51, 215, 127, 98, 107, 27, 74, 136, 229, 204,
    221, 12, 134, 109, 84, 205, 171, 143, 68, 216, 149, 141, 104, 95, 214, 46,
], dtype=np.int32)

_P_ONEHOT = np.zeros((_N, _N), dtype=np.float32)
_P_ONEHOT[_PERM, np.arange(_N)] = 1.0


def _tc_body(lam_ref, y_ref, p_ref, o_ref, m_ref):
    @pl.when(pl.program_id(0) == 0)
    def _():
        lam = lam_ref[0]
        row = lax.broadcasted_iota(jnp.int32, (_N, _N), 0)
        col = lax.broadcasted_iota(jnp.int32, (_N, _N), 1)
        eye = (row == col).astype(jnp.float32)
        m_ref[...] = (1.0 - lam) * eye + lam * p_ref[...]

    o_ref[...] = jnp.dot(y_ref[...], m_ref[...],
                         preferred_element_type=jnp.float32)


@functools.cache
def _tc_call():
    return pl.pallas_call(
        _tc_body,
        grid=(_R // _BR,),
        in_specs=[
            pl.BlockSpec(memory_space=pltpu.SMEM),
            pl.BlockSpec((_BR, _N), lambda i: (i, 0)),
            pl.BlockSpec((_N, _N), lambda i: (0, 0)),
        ],
        out_specs=pl.BlockSpec((_BR, _N), lambda i: (i, 0)),
        out_shape=jax.ShapeDtypeStruct((_R, _N), jnp.float32),
        scratch_shapes=[pltpu.VMEM((_N, _N), jnp.float32)],
    )


def _sc_body(lab_hbm, perm_hbm, olab_hbm, perm_v, olab_v, sl):
    cid = lax.axis_index("c")
    sid = lax.axis_index("s")
    wid = sid * _NC + cid

    @pl.when(wid == 0)
    def _():
        pltpu.sync_copy(perm_hbm, perm_v)
        for h in range(4):
            pltpu.async_copy(
                lab_hbm.at[perm_v.at[pl.ds(h * 64, 64)]], olab_v, sl).wait()
            pltpu.sync_copy(olab_v, olab_hbm.at[pl.ds(h * 64, 64)])


@functools.cache
def _sc_call():
    return pl.kernel(
        _sc_body,
        out_type=[jax.ShapeDtypeStruct((_N, 128), jnp.int32)],
        mesh=plsc.VectorSubcoreMesh(core_axis_name="c", subcore_axis_name="s",
                                    num_cores=_NC, num_subcores=_NS),
        scratch_types=[
            pltpu.VMEM((_N,), jnp.int32),
            pltpu.VMEM((64, 128), jnp.int32),
            pltpu.SemaphoreType.DMA,
        ],
    )


def kernel(x, labels, lambda_):
    y = x.transpose(1, 2, 3, 0).reshape(_R, _N)
    lam1 = lambda_.astype(jnp.float32).reshape(1)
    out2d = _tc_call()(lam1, y, jnp.asarray(_P_ONEHOT))
    out = out2d.reshape(3, 224, 224, _N).transpose(3, 0, 1, 2)

    lab2d = jnp.broadcast_to(labels.astype(jnp.int32)[:, None], (_N, 128))
    (olab,) = _sc_call()(lab2d, jnp.asarray(_PERM))
    return out, labels, olab[:, 0].astype(labels.dtype)
